# baseline (speedup 1.0000x reference)
import functools

import jax
import jax.numpy as jnp
from jax.experimental import pallas as pl
from jax.experimental.pallas import tpu as pltpu

_LAYERS = [(32, 3, True), (64, 3, True), (128, 3, False), (64, 1, False),
           (128, 3, True), (256, 3, False), (128, 1, False), (256, 3, True),
           (512, 3, False), (256, 1, False), (512, 3, False), (256, 1, False),
           (512, 3, True), (1024, 3, False), (512, 1, False), (1024, 3, False),
           (512, 1, False), (1024, 3, False)]
_CLS = 20
_BB = 2
_S = 7


def _bn_scale_bias(p):
    s = p['gamma'] * jax.lax.rsqrt(p['var'] + 1e-5)
    b = p['beta'] - p['mean'] * s
    return jnp.stack([s, b])


def _leaky(y):
    return jnp.where(y >= 0, y, 0.1 * y)


def _conv0_body(x_ref, w_ref, sb_ref, o_ref, *, RC, Wc, Cout):
    h = pl.program_id(1)
    r0 = h * RC
    ii = jax.lax.broadcasted_iota(jnp.int32, (Wc - 1, Wc // 2), 0)
    jj = jax.lax.broadcasted_iota(jnp.int32, (Wc - 1, Wc // 2), 1)
    E = (ii == 2 * jj).astype(jnp.float32)

    def co_body(co, carry):
        acc = jnp.zeros((RC, Wc), jnp.float32)
        for ci in range(3):
            chunk = x_ref[0, ci, pl.ds(r0, RC + 2), :]
            chunk = chunk.astype(jnp.bfloat16).astype(jnp.float32)
            for dy in range(3):
                row = chunk[dy:dy + RC]
                for dx in range(3):
                    acc = acc + row[:, dx:dx + Wc] * w_ref[co, ci * 9 + dy * 3 + dx]
        y = acc * sb_ref[0, co] + sb_ref[1, co]
        y = jnp.where(y >= 0, y, 0.1 * y)
        y = jnp.max(y.reshape(RC // 2, 2, Wc), axis=1)
        y = jnp.maximum(y[:, :Wc - 1], y[:, 1:])
        o_ref[0, pl.ds(co, 1)] = jnp.dot(y, E,
                                         preferred_element_type=jnp.float32, precision=jax.lax.Precision.HIGHEST)[None]
        return carry

    jax.lax.fori_loop(0, Cout, co_body, 0)


def _conv0(x, w, sb):
    N, _, Hc, Wc = x.shape
    Cout = w.shape[0]
    xp = jnp.pad(x, ((0, 0), (0, 0), (1, 1), (1, 1)))
    BH = 64
    HT = Hc // BH
    body = functools.partial(_conv0_body, RC=BH, Wc=Wc, Cout=Cout)
    return pl.pallas_call(
        body,
        grid=(N, HT),
        in_specs=[
            pl.BlockSpec((1, 3, Hc + 2, Wc + 2), lambda n, h: (n, 0, 0, 0)),
            pl.BlockSpec(memory_space=pltpu.SMEM),
            pl.BlockSpec(memory_space=pltpu.SMEM),
        ],
        out_specs=pl.BlockSpec((1, Cout, BH // 2, Wc // 2),
                               lambda n, h: (n, 0, h, 0)),
        out_shape=jax.ShapeDtypeStruct((N, Cout, Hc // 2, Wc // 2),
                                       jnp.float32),
    )(xp, w.reshape(Cout, 27).astype(jnp.bfloat16).astype(jnp.float32), sb)


def _conv3_body(x_ref, w_ref, sb_ref, o_ref, *, RC, Wc, Cin, BCO, pool):
    h = pl.program_id(1)
    r0 = h * RC
    acc = None
    for dy in range(3):
        rows = x_ref[0, pl.ds(r0 + dy, RC)]
        xcat = jnp.concatenate([rows[:, dx:dx + Wc, :] for dx in range(3)],
                               axis=2)
        part = jnp.dot(xcat.reshape(RC * Wc, 3 * Cin), w_ref[dy],
                       preferred_element_type=jnp.float32)
        acc = part if acc is None else acc + part
    y = _leaky(acc * sb_ref[0:1, :] + sb_ref[1:2, :])
    if pool:
        y = jnp.max(y.reshape(RC // 2, 2, Wc, BCO), axis=1)
        y = jnp.max(y.reshape(RC // 2, Wc // 2, 2, BCO), axis=2)
        o_ref[0] = y
    else:
        o_ref[0] = y.reshape(RC, Wc, BCO)


def _conv3_halo_body(xa_ref, xb_ref, w_ref, sb_ref, o_ref, *, RC, Wc, Cin,
                     BCO, pool):
    xa = xa_ref[0]
    acc = None
    for dy in range(3):
        if dy == 0:
            rows = xa
        else:
            rows = jnp.concatenate([xa[dy:], xb_ref[0, :dy]], axis=0)
        xcat = jnp.concatenate([rows[:, dx:dx + Wc, :] for dx in range(3)],
                               axis=2)
        part = jnp.dot(xcat.reshape(RC * Wc, 3 * Cin), w_ref[dy],
                       preferred_element_type=jnp.float32)
        acc = part if acc is None else acc + part
    y = _leaky(acc * sb_ref[0:1, :] + sb_ref[1:2, :])
    if pool:
        y = jnp.max(y.reshape(RC // 2, 2, Wc, BCO), axis=1)
        y = jnp.max(y.reshape(RC // 2, Wc // 2, 2, BCO), axis=2)
        o_ref[0] = y
    else:
        o_ref[0] = y.reshape(RC, Wc, BCO)


def _conv3x3(x, wT, sb, pool):
    N, Hc, Wc, Cin = x.shape
    Cout = wT.shape[2]
    xp = jnp.pad(x, ((0, 0), (1, 1), (1, 1), (0, 0)))
    cands = [d for d in range(Hc, 0, -1)
             if Hc % d == 0 and (not pool or d % 2 == 0)]
    BH = next((d for d in cands
               if d * Wc * max(3 * Cin, 128) * 4 <= 2.5e6), cands[-1])
    HT = Hc // BH
    wbytes = wT.size * 4
    CT = 4 if wbytes > 20e6 else (2 if wbytes > 10e6 else 1)
    BCO = Cout // CT
    BHO, Ho, Wo = (BH // 2, Hc // 2, Wc // 2) if pool else (BH, Hc, Wc)
    halo = Hc * (Wc + 2) * max(Cin, 128) * 4 > 12e6
    if halo:
        xp = jnp.pad(xp, ((0, 0), (0, (HT + 1) * BH - (Hc + 2)), (0, 0),
                          (0, 0)))
        body = functools.partial(_conv3_halo_body, RC=BH, Wc=Wc, Cin=Cin,
                                 BCO=BCO, pool=pool)
        in_specs = [
            pl.BlockSpec((1, BH, Wc + 2, Cin), lambda n, h, c: (n, h, 0, 0)),
            pl.BlockSpec((1, BH, Wc + 2, Cin),
                         lambda n, h, c: (n, h + 1, 0, 0)),
        ]
        args = (xp, xp, wT, sb)
    else:
        body = functools.partial(_conv3_body, RC=BH, Wc=Wc, Cin=Cin, BCO=BCO,
                                 pool=pool)
        in_specs = [
            pl.BlockSpec((1, Hc + 2, Wc + 2, Cin),
                         lambda n, h, c: (n, 0, 0, 0)),
        ]
        args = (xp, wT, sb)
    in_specs += [
        pl.BlockSpec((3, 3 * Cin, BCO), lambda n, h, c: (0, 0, c)),
        pl.BlockSpec((2, BCO), lambda n, h, c: (0, c)),
    ]
    return pl.pallas_call(
        body,
        grid=(N, HT, CT),
        in_specs=in_specs,
        out_specs=pl.BlockSpec((1, BHO, Wo, BCO), lambda n, h, c: (n, h, 0, c)),
        out_shape=jax.ShapeDtypeStruct((N, Ho, Wo, Cout), jnp.float32),
    )(*args)


def _mm1x1_body(x_ref, w_ref, sb_ref, o_ref):
    y = jnp.dot(x_ref[...], w_ref[...], preferred_element_type=jnp.float32)
    o_ref[...] = _leaky(y * sb_ref[0:1, :] + sb_ref[1:2, :])


def _conv1x1(x, wT, sb):
    N, H, W, Cin = x.shape
    Cout = wT.shape[1]
    R = N * H * W
    BM = next((c for c in (3136, 1568, 784, 392, 196)
               if R % c == 0 and c * Cin * 4 <= 2.5e6), R)
    xr = x.reshape(R, Cin)
    out = pl.pallas_call(
        _mm1x1_body,
        grid=(R // BM,),
        in_specs=[
            pl.BlockSpec((BM, Cin), lambda m: (m, 0)),
            pl.BlockSpec((Cin, Cout), lambda m: (0, 0)),
            pl.BlockSpec((2, Cout), lambda m: (0, 0)),
        ],
        out_specs=pl.BlockSpec((BM, Cout), lambda m: (m, 0)),
        out_shape=jax.ShapeDtypeStruct((R, Cout), jnp.float32),
    )(xr, wT, sb)
    return out.reshape(N, H, W, Cout)


def _local0_body(x_ref, w_ref, sb_ref, o_ref, *, C4, BCO):
    acc = None
    for bi in range(2):
        for bj in range(2):
            xs = x_ref[:, bi:bi + _S, bj:bj + _S, :].reshape(2 * _S * _S, C4)
            part = jnp.dot(xs, w_ref[bi * 2 + bj],
                           preferred_element_type=jnp.float32)
            acc = part if acc is None else acc + part
    y = _leaky(acc * sb_ref[0:1, :] + sb_ref[1:2, :])
    o_ref[...] = y.reshape(2, _S, _S, BCO)


def _local0(x, w, sb):
    N, H, W, Cin = x.shape
    Cout = w.shape[0]
    C4 = 4 * Cin
    x2 = x.reshape(N, _S, 2, _S, 2, Cin).transpose(0, 1, 3, 2, 4, 5)
    x2 = x2.reshape(N, _S, _S, C4)
    x2p = jnp.pad(x2, ((0, 0), (1, 0), (1, 0), (0, 0)))
    W2 = jnp.zeros((2, 2, 2, 2, Cin, Cout), jnp.float32)
    for dy in range(3):
        for dx in range(3):
            bi, py = (dy + 1) // 2, (dy + 1) % 2
            bj, px = (dx + 1) // 2, (dx + 1) % 2
            W2 = W2.at[bi, bj, py, px].set(w[:, :, dy, dx].T)
    W2 = W2.transpose(0, 1, 2, 3, 4, 5).reshape(2, 2, C4, Cout)
    W2 = W2.reshape(4, C4, Cout)
    CT = 4
    BCO = Cout // CT
    body = functools.partial(_local0_body, C4=C4, BCO=BCO)
    return pl.pallas_call(
        body,
        grid=(CT,),
        in_specs=[
            pl.BlockSpec((N, _S + 1, _S + 1, C4), lambda c: (0, 0, 0, 0)),
            pl.BlockSpec((4, C4, BCO), lambda c: (0, 0, c)),
            pl.BlockSpec((2, BCO), lambda c: (0, c)),
        ],
        out_specs=pl.BlockSpec((N, _S, _S, BCO), lambda c: (0, 0, 0, c)),
        out_shape=jax.ShapeDtypeStruct((N, _S, _S, Cout), jnp.float32),
    )(x2p, W2, sb)


def _fc_body(a_ref, w_ref, b_ref, o_ref, *, KT, leaky):
    k = pl.program_id(1)

    @pl.when(k == 0)
    def _init():
        o_ref[...] = jnp.zeros_like(o_ref)

    o_ref[...] += jax.lax.dot_general(
        a_ref[...], w_ref[...], (((1,), (1,)), ((), ())),
        preferred_element_type=jnp.float32)

    @pl.when(k == KT - 1)
    def _fin():
        y = o_ref[...] + b_ref[...]
        o_ref[...] = _leaky(y) if leaky else y


def _fc(a, W, b, BO, BK, leaky):
    M, K = a.shape
    O = W.shape[0]
    OT, KT = O // BO, K // BK
    body = functools.partial(_fc_body, KT=KT, leaky=leaky)
    return pl.pallas_call(
        body,
        grid=(OT, KT),
        in_specs=[
            pl.BlockSpec((M, BK), lambda o, k: (0, k)),
            pl.BlockSpec((BO, BK), lambda o, k: (o, k)),
            pl.BlockSpec((1, BO), lambda o, k: (0, o)),
        ],
        out_specs=pl.BlockSpec((M, BO), lambda o, k: (0, o)),
        out_shape=jax.ShapeDtypeStruct((M, O), jnp.float32),
    )(a, W, b.reshape(1, O))


def kernel(x, target, params):
    del target
    N = x.shape[0]
    out = jnp.transpose(x, (0, 2, 3, 1))
    for p, (out_c, k, pool) in zip(params['darknet'], _LAYERS):
        sb = _bn_scale_bias(p)
        w = p['w']
        if k == 3:
            cin = w.shape[1]
            wT = w.transpose(2, 3, 1, 0).reshape(3, 3 * cin, out_c)
            out = _conv3x3(out, wT, sb, pool)
        else:
            wT = w.reshape(out_c, w.shape[1]).T
            out = _conv1x1(out, wT, sb)
    lp = params['local']
    out = _local0(out, lp[0]['w'], _bn_scale_bias(lp[0]))
    for p in lp[1:]:
        cin = p['w'].shape[1]
        wT = p['w'].transpose(2, 3, 1, 0).reshape(3, 3 * cin, 1024)
        out = _conv3x3(out, wT, _bn_scale_bias(p), False)
    flat = jnp.transpose(out, (0, 3, 1, 2)).reshape(N, -1)
    h = _fc(flat, params['reg_W'], params['reg_b'], BO=256, BK=1792,
            leaky=True)
    headW = jnp.concatenate([params['cls_W'], params['resp_W'],
                             params['off_W'],
                             jnp.zeros((66, 4096), jnp.float32)], axis=0)
    headb = jnp.concatenate([params['cls_b'], params['resp_b'],
                             params['off_b'],
                             jnp.zeros((66,), jnp.float32)], axis=0)
    hy = _fc(h, headW, headb, BO=256, BK=4096, leaky=False)
    hy = hy[:, :1470]
    n_cls = _CLS * _S * _S
    n_resp = _BB * _S * _S
    pred_cls = hy[:, :n_cls].reshape(N, _CLS, _S, _S)
    pred_response = hy[:, n_cls:n_cls + n_resp].reshape(N, _BB, _S, _S)
    pred_bbox = hy[:, n_cls + n_resp:].reshape(N, _BB * 4, _S, _S)
    return (pred_cls, pred_response, pred_bbox)


# speedup vs baseline: 1.0916x; 1.0916x over previous
import functools

import jax
import jax.numpy as jnp
from jax.experimental import pallas as pl
from jax.experimental.pallas import tpu as pltpu

_LAYERS = [(32, 3, True), (64, 3, True), (128, 3, False), (64, 1, False),
           (128, 3, True), (256, 3, False), (128, 1, False), (256, 3, True),
           (512, 3, False), (256, 1, False), (512, 3, False), (256, 1, False),
           (512, 3, True), (1024, 3, False), (512, 1, False), (1024, 3, False),
           (512, 1, False), (1024, 3, False)]
_CLS = 20
_BB = 2
_S = 7


def _bn_scale_bias(p):
    s = p['gamma'] * jax.lax.rsqrt(p['var'] + 1e-5)
    b = p['beta'] - p['mean'] * s
    return jnp.stack([s, b])


def _leaky(y):
    return jnp.where(y >= 0, y, 0.1 * y)


def _conv0_body(x_ref, w_ref, sb_ref, o_ref, *, RC, Wc, Cout):
    h = pl.program_id(1)
    r0 = h * RC
    ii = jax.lax.broadcasted_iota(jnp.int32, (Wc - 1, Wc // 2), 0)
    jj = jax.lax.broadcasted_iota(jnp.int32, (Wc - 1, Wc // 2), 1)
    E = (ii == 2 * jj).astype(jnp.float32)

    def co_body(co, carry):
        acc = jnp.zeros((RC, Wc), jnp.float32)
        for ci in range(3):
            chunk = x_ref[0, ci, pl.ds(r0, RC + 2), :]
            chunk = chunk.astype(jnp.bfloat16).astype(jnp.float32)
            for dy in range(3):
                row = chunk[dy:dy + RC]
                for dx in range(3):
                    acc = acc + row[:, dx:dx + Wc] * w_ref[co, ci * 9 + dy * 3 + dx]
        y = acc * sb_ref[0, co] + sb_ref[1, co]
        y = jnp.where(y >= 0, y, 0.1 * y)
        y = jnp.max(y.reshape(RC // 2, 2, Wc), axis=1)
        y = jnp.maximum(y[:, :Wc - 1], y[:, 1:])
        o_ref[0, pl.ds(co, 1)] = jnp.dot(y, E,
                                         preferred_element_type=jnp.float32, precision=jax.lax.Precision.HIGHEST)[None]
        return carry

    jax.lax.fori_loop(0, Cout, co_body, 0)


def _conv0(x, w, sb):
    N, _, Hc, Wc = x.shape
    Cout = w.shape[0]
    xp = jnp.pad(x, ((0, 0), (0, 0), (1, 1), (1, 1)))
    BH = 64
    HT = Hc // BH
    body = functools.partial(_conv0_body, RC=BH, Wc=Wc, Cout=Cout)
    return pl.pallas_call(
        body,
        grid=(N, HT),
        in_specs=[
            pl.BlockSpec((1, 3, Hc + 2, Wc + 2), lambda n, h: (n, 0, 0, 0)),
            pl.BlockSpec(memory_space=pltpu.SMEM),
            pl.BlockSpec(memory_space=pltpu.SMEM),
        ],
        out_specs=pl.BlockSpec((1, Cout, BH // 2, Wc // 2),
                               lambda n, h: (n, 0, h, 0)),
        out_shape=jax.ShapeDtypeStruct((N, Cout, Hc // 2, Wc // 2),
                                       jnp.float32),
    )(xp, w.reshape(Cout, 27).astype(jnp.bfloat16).astype(jnp.float32), sb)


def _conv3_body(x_ref, w_ref, sb_ref, o_ref, *, RC, Wc, Cin, BCO, pool):
    h = pl.program_id(1)
    r0 = h * RC
    acc = None
    for dy in range(3):
        rows = x_ref[0, pl.ds(r0 + dy, RC)]
        xcat = jnp.concatenate([rows[:, dx:dx + Wc, :] for dx in range(3)],
                               axis=2)
        part = jnp.dot(xcat.reshape(RC * Wc, 3 * Cin), w_ref[dy],
                       preferred_element_type=jnp.float32)
        acc = part if acc is None else acc + part
    y = _leaky(acc * sb_ref[0:1, :] + sb_ref[1:2, :]).astype(jnp.bfloat16)
    if pool:
        y = jnp.max(y.reshape(RC // 2, 2, Wc, BCO), axis=1)
        y = jnp.max(y.reshape(RC // 2, Wc // 2, 2, BCO), axis=2)
        o_ref[0] = y
    else:
        o_ref[0] = y.reshape(RC, Wc, BCO)


def _conv3_halo_body(xa_ref, xb_ref, w_ref, sb_ref, o_ref, *, RC, Wc, Cin,
                     BCO, pool):
    xa = xa_ref[0]
    acc = None
    for dy in range(3):
        if dy == 0:
            rows = xa
        else:
            rows = jnp.concatenate([xa[dy:], xb_ref[0, :dy]], axis=0)
        xcat = jnp.concatenate([rows[:, dx:dx + Wc, :] for dx in range(3)],
                               axis=2)
        part = jnp.dot(xcat.reshape(RC * Wc, 3 * Cin), w_ref[dy],
                       preferred_element_type=jnp.float32)
        acc = part if acc is None else acc + part
    y = _leaky(acc * sb_ref[0:1, :] + sb_ref[1:2, :]).astype(jnp.bfloat16)
    if pool:
        y = jnp.max(y.reshape(RC // 2, 2, Wc, BCO), axis=1)
        y = jnp.max(y.reshape(RC // 2, Wc // 2, 2, BCO), axis=2)
        o_ref[0] = y
    else:
        o_ref[0] = y.reshape(RC, Wc, BCO)


def _conv3x3(x, wT, sb, pool):
    N, Hc, Wc, Cin = x.shape
    Cout = wT.shape[2]
    xp = jnp.pad(x, ((0, 0), (1, 1), (1, 1), (0, 0)))
    cands = [d for d in range(Hc, 0, -1)
             if Hc % d == 0 and (not pool or d % 2 == 0)]
    budget = 2.5e6 if 3 * Cin >= 128 else 5.0e6
    BH = next((d for d in cands
               if d * Wc * max(3 * Cin, 128) * 4 <= budget), cands[-1])
    HT = Hc // BH
    wbytes = wT.size * 4
    CT = 4 if wbytes > 20e6 else (2 if wbytes > 10e6 else 1)
    BCO = Cout // CT
    BHO, Ho, Wo = (BH // 2, Hc // 2, Wc // 2) if pool else (BH, Hc, Wc)
    halo = Hc * (Wc + 2) * max(Cin, 128) * 4 > 12e6
    if halo:
        xp = jnp.pad(xp, ((0, 0), (0, (HT + 1) * BH - (Hc + 2)), (0, 0),
                          (0, 0)))
        body = functools.partial(_conv3_halo_body, RC=BH, Wc=Wc, Cin=Cin,
                                 BCO=BCO, pool=pool)
        in_specs = [
            pl.BlockSpec((1, BH, Wc + 2, Cin), lambda n, h, c: (n, h, 0, 0)),
            pl.BlockSpec((1, BH, Wc + 2, Cin),
                         lambda n, h, c: (n, h + 1, 0, 0)),
        ]
        args = (xp, xp, wT, sb)
    else:
        body = functools.partial(_conv3_body, RC=BH, Wc=Wc, Cin=Cin, BCO=BCO,
                                 pool=pool)
        in_specs = [
            pl.BlockSpec((1, Hc + 2, Wc + 2, Cin),
                         lambda n, h, c: (n, 0, 0, 0)),
        ]
        args = (xp, wT, sb)
    in_specs += [
        pl.BlockSpec((3, 3 * Cin, BCO), lambda n, h, c: (0, 0, c)),
        pl.BlockSpec((2, BCO), lambda n, h, c: (0, c)),
    ]
    return pl.pallas_call(
        body,
        grid=(N, HT, CT),
        in_specs=in_specs,
        out_specs=pl.BlockSpec((1, BHO, Wo, BCO), lambda n, h, c: (n, h, 0, c)),
        out_shape=jax.ShapeDtypeStruct((N, Ho, Wo, Cout), jnp.bfloat16),
    )(*args)


def _mm1x1_body(x_ref, w_ref, sb_ref, o_ref):
    y = jnp.dot(x_ref[...], w_ref[...], preferred_element_type=jnp.float32)
    o_ref[...] = _leaky(y * sb_ref[0:1, :] + sb_ref[1:2, :]).astype(o_ref.dtype)


def _conv1x1(x, wT, sb):
    N, H, W, Cin = x.shape
    Cout = wT.shape[1]
    R = N * H * W
    BM = next((c for c in (3136, 1568, 784, 392, 196)
               if R % c == 0 and c * Cin * 4 <= 2.5e6), R)
    xr = x.reshape(R, Cin)
    out = pl.pallas_call(
        _mm1x1_body,
        grid=(R // BM,),
        in_specs=[
            pl.BlockSpec((BM, Cin), lambda m: (m, 0)),
            pl.BlockSpec((Cin, Cout), lambda m: (0, 0)),
            pl.BlockSpec((2, Cout), lambda m: (0, 0)),
        ],
        out_specs=pl.BlockSpec((BM, Cout), lambda m: (m, 0)),
        out_shape=jax.ShapeDtypeStruct((R, Cout), jnp.bfloat16),
    )(xr, wT, sb)
    return out.reshape(N, H, W, Cout)


def _local0_body(x_ref, w_ref, sb_ref, o_ref, *, C4, BCO):
    acc = None
    for bi in range(2):
        for bj in range(2):
            xs = x_ref[:, bi:bi + _S, bj:bj + _S, :].reshape(2 * _S * _S, C4)
            part = jnp.dot(xs, w_ref[bi * 2 + bj],
                           preferred_element_type=jnp.float32)
            acc = part if acc is None else acc + part
    y = _leaky(acc * sb_ref[0:1, :] + sb_ref[1:2, :])
    o_ref[...] = y.reshape(2, _S, _S, BCO).astype(o_ref.dtype)


def _local0(x, w, sb):
    N, H, W, Cin = x.shape
    Cout = w.shape[0]
    C4 = 4 * Cin
    x2 = x.reshape(N, _S, 2, _S, 2, Cin).transpose(0, 1, 3, 2, 4, 5)
    x2 = x2.reshape(N, _S, _S, C4)
    x2p = jnp.pad(x2, ((0, 0), (1, 0), (1, 0), (0, 0)))
    w = w.astype(jnp.bfloat16)
    W2 = jnp.zeros((2, 2, 2, 2, Cin, Cout), jnp.bfloat16)
    for dy in range(3):
        for dx in range(3):
            bi, py = (dy + 1) // 2, (dy + 1) % 2
            bj, px = (dx + 1) // 2, (dx + 1) % 2
            W2 = W2.at[bi, bj, py, px].set(w[:, :, dy, dx].T)
    W2 = W2.transpose(0, 1, 2, 3, 4, 5).reshape(2, 2, C4, Cout)
    W2 = W2.reshape(4, C4, Cout)
    CT = 4
    BCO = Cout // CT
    body = functools.partial(_local0_body, C4=C4, BCO=BCO)
    return pl.pallas_call(
        body,
        grid=(CT,),
        in_specs=[
            pl.BlockSpec((N, _S + 1, _S + 1, C4), lambda c: (0, 0, 0, 0)),
            pl.BlockSpec((4, C4, BCO), lambda c: (0, 0, c)),
            pl.BlockSpec((2, BCO), lambda c: (0, c)),
        ],
        out_specs=pl.BlockSpec((N, _S, _S, BCO), lambda c: (0, 0, 0, c)),
        out_shape=jax.ShapeDtypeStruct((N, _S, _S, Cout), jnp.bfloat16),
    )(x2p, W2, sb)


def _fc_body(a_ref, w_ref, b_ref, o_ref, *, KT, leaky):
    k = pl.program_id(1)

    @pl.when(k == 0)
    def _init():
        o_ref[...] = jnp.zeros_like(o_ref)

    o_ref[...] += jax.lax.dot_general(
        a_ref[...], w_ref[...], (((1,), (1,)), ((), ())),
        preferred_element_type=jnp.float32)

    @pl.when(k == KT - 1)
    def _fin():
        y = o_ref[...] + b_ref[...]
        o_ref[...] = _leaky(y) if leaky else y


def _fc(a, W, b, BO, BK, leaky):
    M, K = a.shape
    O = W.shape[0]
    OT, KT = O // BO, K // BK
    body = functools.partial(_fc_body, KT=KT, leaky=leaky)
    return pl.pallas_call(
        body,
        grid=(OT, KT),
        in_specs=[
            pl.BlockSpec((M, BK), lambda o, k: (0, k)),
            pl.BlockSpec((BO, BK), lambda o, k: (o, k)),
            pl.BlockSpec((1, BO), lambda o, k: (0, o)),
        ],
        out_specs=pl.BlockSpec((M, BO), lambda o, k: (0, o)),
        out_shape=jax.ShapeDtypeStruct((M, O), jnp.float32),
    )(a, W, b.reshape(1, O))


def kernel(x, target, params):
    del target
    N = x.shape[0]
    out = jnp.transpose(x.astype(jnp.bfloat16), (0, 2, 3, 1))
    for p, (out_c, k, pool) in zip(params['darknet'], _LAYERS):
        sb = _bn_scale_bias(p)
        w = p['w']
        if k == 3:
            cin = w.shape[1]
            wT = w.astype(jnp.bfloat16).transpose(2, 3, 1, 0)
            wT = wT.reshape(3, 3 * cin, out_c)
            out = _conv3x3(out, wT, sb, pool)
        else:
            wT = w.astype(jnp.bfloat16).reshape(out_c, w.shape[1]).T
            out = _conv1x1(out, wT, sb)
    lp = params['local']
    out = _local0(out, lp[0]['w'], _bn_scale_bias(lp[0]))
    for p in lp[1:]:
        cin = p['w'].shape[1]
        wT = p['w'].astype(jnp.bfloat16).transpose(2, 3, 1, 0)
        wT = wT.reshape(3, 3 * cin, 1024)
        out = _conv3x3(out, wT, _bn_scale_bias(p), False)
    flat = jnp.transpose(out, (0, 3, 1, 2)).reshape(N, -1).astype(jnp.float32)
    h = _fc(flat, params['reg_W'], params['reg_b'], BO=256, BK=1792,
            leaky=True)
    headW = jnp.concatenate([params['cls_W'], params['resp_W'],
                             params['off_W'],
                             jnp.zeros((66, 4096), jnp.float32)], axis=0)
    headb = jnp.concatenate([params['cls_b'], params['resp_b'],
                             params['off_b'],
                             jnp.zeros((66,), jnp.float32)], axis=0)
    hy = _fc(h, headW, headb, BO=256, BK=4096, leaky=False)
    hy = hy[:, :1470]
    n_cls = _CLS * _S * _S
    n_resp = _BB * _S * _S
    pred_cls = hy[:, :n_cls].reshape(N, _CLS, _S, _S)
    pred_response = hy[:, n_cls:n_cls + n_resp].reshape(N, _BB, _S, _S)
    pred_bbox = hy[:, n_cls + n_resp:].reshape(N, _BB * 4, _S, _S)
    return (pred_cls, pred_response, pred_bbox)
